# baseline (speedup 1.0000x reference)
import functools

import jax
import jax.numpy as jnp
from jax import lax
from jax.experimental import pallas as pl
from jax.experimental.pallas import tpu as pltpu
from jax.experimental.pallas import tpu_sc as plsc

_N = 10000
_E = 160000
_D = 256
_R = 16
_H = _D // 2
_C = 128
_NSUB = 16
_EPS = _E // _NSUB
_NFULL = _EPS // _C
_ETAIL = _EPS - _NFULL * _C
_CPB = 6
_B = _CPB * _C
_NBLK = _NFULL // _CPB
_SROWS = 624
_TAIL = _N - _NSUB * _SROWS
_WB = 24


def _matmul_body(x_ref, w_ref, o_ref):
    o_ref[...] = jnp.dot(x_ref[...], w_ref[0],
                         preferred_element_type=jnp.float32)[None]


def _per_rel_matmul(x, w3):
    return pl.pallas_call(
        _matmul_body,
        grid=(2 * _R,),
        in_specs=[
            pl.BlockSpec((_N, _D), lambda q: (0, 0)),
            pl.BlockSpec((1, _D, _H), lambda q: (q, 0, 0)),
        ],
        out_specs=pl.BlockSpec((1, _N, _H), lambda q: (q, 0, 0)),
        out_shape=jax.ShapeDtypeStruct((2 * _R, _N, _H), jnp.float32),
    )(x.astype(jnp.bfloat16), w3.astype(jnp.bfloat16))


def _sc_kernel_body(table_hbm, src_hbm, dst_hbm, rel_hbm, norm_hbm, out_hbm,
                    src_b, dst_b, rel_b, norm_b,
                    rows0, rows1, gidx0, gidx1, dstb0, dstb1, gidx_t, dstb_t,
                    wb, acc, gsem0, gsem1, ssem0, ssem1):
    cid = lax.axis_index("c")
    sid = lax.axis_index("s")
    rows = (rows0, rows1)
    gidx = (gidx0, gidx1)
    dstb = (dstb0, dstb1)
    gsem = (gsem0, gsem1)
    ssem = (ssem0, ssem1)

    zeros16 = jnp.zeros((16,), jnp.float32)

    def _zero_row(r, _):
        for g in range(_H // 16):
            wb[r, pl.ds(g * 16, 16)] = zeros16
        return 0

    lax.fori_loop(0, _WB, _zero_row, 0)
    base_row = sid * _SROWS
    for t in range(_SROWS // _WB):
        pltpu.sync_copy(wb, acc.at[pl.ds(base_row + t * _WB, _WB)])

    @pl.when(sid == 0)
    def _zero_tail():
        pltpu.sync_copy(wb.at[pl.ds(0, _TAIL)],
                        acc.at[pl.ds(_NSUB * _SROWS, _TAIL)])

    plsc.subcore_barrier()

    ebase = sid * _EPS

    def _prep(j, slot):
        for g in range(_C // 16):
            sl = pl.ds(g * 16, 16)
            msl = pl.ds(j * _C + g * 16, 16)
            gidx[slot][sl] = (rel_b[msl] * 2 + cid) * _N + src_b[msl]
            dstb[slot][sl] = dst_b[msl]
        pltpu.make_async_copy(table_hbm.at[gidx[slot]], rows[slot],
                              gsem[slot]).start()

    def _scale(j, slot):
        rv = rows[slot]

        def _grp(g, _):
            norm16 = norm_b[pl.ds(j * _C + g * 16, 16)]
            for lane in range(16):
                nrm = norm16[lane]
                jj = g * 16 + lane
                for q in range(_H // 16):
                    sl = pl.ds(q * 16, 16)
                    rv[jj, sl] = rv[jj, sl] * nrm
            return 0

        lax.fori_loop(0, _C // 16, _grp, 0)

    def _scatter_wait(s):
        pltpu.make_async_copy(rows[s], acc.at[dstb[s]], ssem[s]).wait()

    def _block(b, _):
        bbase = ebase + b * _B
        pltpu.sync_copy(src_hbm.at[pl.ds(bbase, _B)], src_b)
        pltpu.sync_copy(dst_hbm.at[pl.ds(bbase, _B)], dst_b)
        pltpu.sync_copy(rel_hbm.at[pl.ds(bbase, _B)], rel_b)
        pltpu.sync_copy(norm_hbm.at[pl.ds(bbase, _B)], norm_b)

        @pl.when(b > 0)
        def _():
            _scatter_wait(0)

        _prep(0, 0)
        for j in range(_CPB):
            slot = j % 2
            if j + 1 < _CPB:
                if j == 0:
                    @pl.when(b > 0)
                    def _():
                        _scatter_wait(1)
                else:
                    _scatter_wait(1 - slot)
                _prep(j + 1, 1 - slot)
            pltpu.make_async_copy(table_hbm.at[gidx[slot]], rows[slot],
                                  gsem[slot]).wait()
            _scale(j, slot)
            pltpu.make_async_copy(rows[slot], acc.at[dstb[slot]],
                                  ssem[slot]).start(add=True)
        return 0

    lax.fori_loop(0, _NBLK, _block, 0)
    _scatter_wait(0)
    _scatter_wait(1)

    toff = ebase + _NFULL * _C
    tsl = pl.ds(0, _ETAIL)
    pltpu.sync_copy(src_hbm.at[pl.ds(toff, _ETAIL)], src_b.at[tsl])
    pltpu.sync_copy(dst_hbm.at[pl.ds(toff, _ETAIL)], dst_b.at[tsl])
    pltpu.sync_copy(rel_hbm.at[pl.ds(toff, _ETAIL)], rel_b.at[tsl])
    pltpu.sync_copy(norm_hbm.at[pl.ds(toff, _ETAIL)], norm_b.at[tsl])
    gidx_t[...] = (rel_b[tsl] * 2 + cid) * _N + src_b[tsl]
    dstb_t[...] = dst_b[tsl]
    pltpu.sync_copy(table_hbm.at[gidx_t], rows0.at[tsl])
    norm16 = norm_b[tsl]
    for lane in range(_ETAIL):
        nrm = norm16[lane]
        for q in range(_H // 16):
            sl = pl.ds(q * 16, 16)
            rows0[lane, sl] = rows0[lane, sl] * nrm
    pltpu.sync_copy(rows0.at[tsl], acc.at[dstb_t], add=True)

    plsc.subcore_barrier()

    def _relu_rows(nrows):
        def _relu_row(r, _):
            for g in range(_H // 16):
                sl = pl.ds(g * 16, 16)
                wb[r, sl] = jnp.maximum(wb[r, sl], 0.0)
            return 0

        lax.fori_loop(0, nrows, _relu_row, 0)

    for t in range(_SROWS // _WB):
        row0 = base_row + t * _WB
        pltpu.sync_copy(acc.at[pl.ds(row0, _WB)], wb)
        _relu_rows(_WB)
        pltpu.sync_copy(wb, out_hbm.at[pl.ds(row0, _WB), pl.ds(cid * _H, _H)])

    @pl.when(sid == 0)
    def _tail_out():
        trow = _NSUB * _SROWS
        pltpu.sync_copy(acc.at[pl.ds(trow, _TAIL)], wb.at[pl.ds(0, _TAIL)])
        _relu_rows(_TAIL)
        pltpu.sync_copy(wb.at[pl.ds(0, _TAIL)],
                        out_hbm.at[pl.ds(trow, _TAIL), pl.ds(cid * _H, _H)])


_sc_scatter = functools.partial(
    pl.kernel,
    out_type=jax.ShapeDtypeStruct((_N, _D), jnp.float32),
    mesh=plsc.VectorSubcoreMesh(core_axis_name="c", subcore_axis_name="s"),
    scratch_types=[
        pltpu.VMEM((_B,), jnp.int32),
        pltpu.VMEM((_B,), jnp.int32),
        pltpu.VMEM((_B,), jnp.int32),
        pltpu.VMEM((_B,), jnp.float32),
        pltpu.VMEM((_C, _H), jnp.float32),
        pltpu.VMEM((_C, _H), jnp.float32),
        pltpu.VMEM((_C,), jnp.int32),
        pltpu.VMEM((_C,), jnp.int32),
        pltpu.VMEM((_C,), jnp.int32),
        pltpu.VMEM((_C,), jnp.int32),
        pltpu.VMEM((_ETAIL,), jnp.int32),
        pltpu.VMEM((_ETAIL,), jnp.int32),
        pltpu.VMEM((_WB, _H), jnp.float32),
        pltpu.VMEM_SHARED((_N, _H), jnp.float32),
        pltpu.SemaphoreType.DMA,
        pltpu.SemaphoreType.DMA,
        pltpu.SemaphoreType.DMA,
        pltpu.SemaphoreType.DMA,
    ],
)(_sc_kernel_body)


def kernel(x, edge_index, edge_type, edge_norm, weight):
    w3 = weight.reshape(_R, _D, 2, _H).transpose(0, 2, 1, 3).reshape(2 * _R, _D, _H)
    per_rel = _per_rel_matmul(x, w3)
    table = per_rel.reshape(2 * _R * _N, _H)
    src = edge_index[0]
    dst = edge_index[1]
    norm = edge_norm.reshape(_E)
    return _sc_scatter(table, src, dst, edge_type, norm)


# speedup vs baseline: 1.1201x; 1.1201x over previous
import functools

import jax
import jax.numpy as jnp
from jax import lax
from jax.experimental import pallas as pl
from jax.experimental.pallas import tpu as pltpu
from jax.experimental.pallas import tpu_sc as plsc

_N = 10000
_E = 160000
_D = 256
_R = 16
_H = _D // 2
_C = 128
_NSUB = 16
_EPS = _E // _NSUB
_NFULL = _EPS // _C
_ETAIL = _EPS - _NFULL * _C
_CPB = 6
_B = _CPB * _C
_NBLK = _NFULL // _CPB
_SROWS = 624
_TAIL = _N - _NSUB * _SROWS
_WB = 24


def _matmul_body(x_ref, w_ref, o_ref):
    o_ref[...] = jnp.dot(x_ref[...], w_ref[0],
                         preferred_element_type=jnp.float32)[None]


def _per_rel_matmul(x, w3):
    return pl.pallas_call(
        _matmul_body,
        grid=(2 * _R,),
        in_specs=[
            pl.BlockSpec((_N, _D), lambda q: (0, 0)),
            pl.BlockSpec((1, _D, _H), lambda q: (q, 0, 0)),
        ],
        out_specs=pl.BlockSpec((1, _N, _H), lambda q: (q, 0, 0)),
        out_shape=jax.ShapeDtypeStruct((2 * _R, _N, _H), jnp.float32),
    )(x.astype(jnp.bfloat16), w3.astype(jnp.bfloat16))


def _sc_kernel_body(table_hbm, src_hbm, dst_hbm, rel_hbm, norm_hbm, out_hbm,
                    src_b0, dst_b0, rel_b0, norm_b0,
                    src_b1, dst_b1, rel_b1, norm_b1,
                    rows0, rows1, gidx0, gidx1, dstb0, dstb1, gidx_t, dstb_t,
                    wb, acc, gsem0, gsem1, ssem0, ssem1, msem0, msem1):
    cid = lax.axis_index("c")
    sid = lax.axis_index("s")
    rows = (rows0, rows1)
    gidx = (gidx0, gidx1)
    dstb = (dstb0, dstb1)
    gsem = (gsem0, gsem1)
    ssem = (ssem0, ssem1)
    meta0 = (src_b0, dst_b0, rel_b0, norm_b0)
    meta1 = (src_b1, dst_b1, rel_b1, norm_b1)
    msem = (msem0, msem1)
    meta_hbm = (src_hbm, dst_hbm, rel_hbm, norm_hbm)

    zeros16 = jnp.zeros((16,), jnp.float32)

    def _zero_row(r, _):
        for g in range(_H // 16):
            wb[r, pl.ds(g * 16, 16)] = zeros16
        return 0

    lax.fori_loop(0, _WB, _zero_row, 0)
    base_row = sid * _SROWS
    for t in range(_SROWS // _WB):
        pltpu.sync_copy(wb, acc.at[pl.ds(base_row + t * _WB, _WB)])

    @pl.when(sid == 0)
    def _zero_tail():
        pltpu.sync_copy(wb.at[pl.ds(0, _TAIL)],
                        acc.at[pl.ds(_NSUB * _SROWS, _TAIL)])

    plsc.subcore_barrier()

    ebase = sid * _EPS

    def _meta_start(b, p):
        bbase = ebase + b * _B
        for hbm, buf in zip(meta_hbm, (meta0, meta1)[p]):
            pltpu.make_async_copy(hbm.at[pl.ds(bbase, _B)], buf,
                                  msem[p]).start()

    def _meta_wait(p):
        for hbm, buf in zip(meta_hbm, (meta0, meta1)[p]):
            pltpu.make_async_copy(hbm.at[pl.ds(0, _B)], buf, msem[p]).wait()

    def _prep(j, slot, meta):
        src_b, dst_b, rel_b, _ = meta
        for g in range(_C // 16):
            sl = pl.ds(g * 16, 16)
            msl = pl.ds(j * _C + g * 16, 16)
            gidx[slot][sl] = (rel_b[msl] * 2 + cid) * _N + src_b[msl]
            dstb[slot][sl] = dst_b[msl]
        pltpu.make_async_copy(table_hbm.at[gidx[slot]], rows[slot],
                              gsem[slot]).start()

    def _scale(j, slot, meta):
        rv = rows[slot]
        norm_b = meta[3]

        def _grp(g, _):
            norm16 = norm_b[pl.ds(j * _C + g * 16, 16)]
            for lane in range(16):
                nrm = norm16[lane]
                jj = g * 16 + lane
                for q in range(_H // 16):
                    sl = pl.ds(q * 16, 16)
                    rv[jj, sl] = rv[jj, sl] * nrm
            return 0

        lax.fori_loop(0, _C // 16, _grp, 0)

    def _scatter_wait(s):
        pltpu.make_async_copy(rows[s], acc.at[dstb[s]], ssem[s]).wait()

    def _block(b, meta):
        @pl.when(b > 0)
        def _():
            _scatter_wait(0)

        _prep(0, 0, meta)
        for j in range(_CPB):
            slot = j % 2
            if j + 1 < _CPB:
                if j == 0:
                    @pl.when(b > 0)
                    def _():
                        _scatter_wait(1)
                else:
                    _scatter_wait(1 - slot)
                _prep(j + 1, 1 - slot, meta)
            pltpu.make_async_copy(table_hbm.at[gidx[slot]], rows[slot],
                                  gsem[slot]).wait()
            _scale(j, slot, meta)
            pltpu.make_async_copy(rows[slot], acc.at[dstb[slot]],
                                  ssem[slot]).start(add=True)

    _meta_start(0, 0)

    def _block_pair(i, _):
        b = 2 * i
        _meta_wait(0)
        _meta_start(b + 1, 1)
        _block(b, meta0)
        _meta_wait(1)
        @pl.when(b + 2 < _NBLK)
        def _():
            _meta_start(b + 2, 0)
        _block(b + 1, meta1)
        return 0

    lax.fori_loop(0, _NBLK // 2, _block_pair, 0)
    _meta_wait(0)
    _block(jnp.int32(_NBLK - 1), meta0)
    _scatter_wait(0)
    _scatter_wait(1)

    toff = ebase + _NFULL * _C
    tsl = pl.ds(0, _ETAIL)
    pltpu.sync_copy(src_hbm.at[pl.ds(toff, _ETAIL)], src_b0.at[tsl])
    pltpu.sync_copy(dst_hbm.at[pl.ds(toff, _ETAIL)], dst_b0.at[tsl])
    pltpu.sync_copy(rel_hbm.at[pl.ds(toff, _ETAIL)], rel_b0.at[tsl])
    pltpu.sync_copy(norm_hbm.at[pl.ds(toff, _ETAIL)], norm_b0.at[tsl])
    gidx_t[...] = (rel_b0[tsl] * 2 + cid) * _N + src_b0[tsl]
    dstb_t[...] = dst_b0[tsl]
    pltpu.sync_copy(table_hbm.at[gidx_t], rows0.at[tsl])
    norm16 = norm_b0[tsl]
    for lane in range(_ETAIL):
        nrm = norm16[lane]
        for q in range(_H // 16):
            sl = pl.ds(q * 16, 16)
            rows0[lane, sl] = rows0[lane, sl] * nrm
    pltpu.sync_copy(rows0.at[tsl], acc.at[dstb_t], add=True)

    plsc.subcore_barrier()

    def _relu_rows(nrows):
        def _relu_row(r, _):
            for g in range(_H // 16):
                sl = pl.ds(g * 16, 16)
                wb[r, sl] = jnp.maximum(wb[r, sl], 0.0)
            return 0

        lax.fori_loop(0, nrows, _relu_row, 0)

    for t in range(_SROWS // _WB):
        row0 = base_row + t * _WB
        pltpu.sync_copy(acc.at[pl.ds(row0, _WB)], wb)
        _relu_rows(_WB)
        pltpu.sync_copy(wb, out_hbm.at[pl.ds(row0, _WB), pl.ds(cid * _H, _H)])

    @pl.when(sid == 0)
    def _tail_out():
        trow = _NSUB * _SROWS
        pltpu.sync_copy(acc.at[pl.ds(trow, _TAIL)], wb.at[pl.ds(0, _TAIL)])
        _relu_rows(_TAIL)
        pltpu.sync_copy(wb.at[pl.ds(0, _TAIL)],
                        out_hbm.at[pl.ds(trow, _TAIL), pl.ds(cid * _H, _H)])


_sc_scatter = functools.partial(
    pl.kernel,
    out_type=jax.ShapeDtypeStruct((_N, _D), jnp.float32),
    mesh=plsc.VectorSubcoreMesh(core_axis_name="c", subcore_axis_name="s"),
    scratch_types=[
        pltpu.VMEM((_B,), jnp.int32),
        pltpu.VMEM((_B,), jnp.int32),
        pltpu.VMEM((_B,), jnp.int32),
        pltpu.VMEM((_B,), jnp.float32),
        pltpu.VMEM((_B,), jnp.int32),
        pltpu.VMEM((_B,), jnp.int32),
        pltpu.VMEM((_B,), jnp.int32),
        pltpu.VMEM((_B,), jnp.float32),
        pltpu.VMEM((_C, _H), jnp.float32),
        pltpu.VMEM((_C, _H), jnp.float32),
        pltpu.VMEM((_C,), jnp.int32),
        pltpu.VMEM((_C,), jnp.int32),
        pltpu.VMEM((_C,), jnp.int32),
        pltpu.VMEM((_C,), jnp.int32),
        pltpu.VMEM((_ETAIL,), jnp.int32),
        pltpu.VMEM((_ETAIL,), jnp.int32),
        pltpu.VMEM((_WB, _H), jnp.float32),
        pltpu.VMEM_SHARED((_N, _H), jnp.float32),
        pltpu.SemaphoreType.DMA,
        pltpu.SemaphoreType.DMA,
        pltpu.SemaphoreType.DMA,
        pltpu.SemaphoreType.DMA,
        pltpu.SemaphoreType.DMA,
        pltpu.SemaphoreType.DMA,
    ],
)(_sc_kernel_body)


def kernel(x, edge_index, edge_type, edge_norm, weight):
    w3 = weight.reshape(_R, _D, 2, _H).transpose(0, 2, 1, 3).reshape(2 * _R, _D, _H)
    per_rel = _per_rel_matmul(x, w3)
    table = per_rel.reshape(2 * _R * _N, _H)
    src = edge_index[0]
    dst = edge_index[1]
    norm = edge_norm.reshape(_E)
    return _sc_scatter(table, src, dst, edge_type, norm)
